# Initial kernel scaffold; baseline (speedup 1.0000x reference)
import jax
import jax.numpy as jnp
from jax.experimental import pallas as pl


def kernel(X, W, rows, cols, vals):
    raise NotImplementedError("write your pallas kernel here")


# speedup vs baseline: 5.2002x; 5.2002x over previous
import functools

import jax
import jax.numpy as jnp
from jax import lax
from jax.experimental import pallas as pl
from jax.experimental.pallas import tpu as pltpu
from jax.experimental.pallas import tpu_sc as plsc

N = 10000
NP = 10240
E = 320000
D = 128

NC = 2
NS = 16
NW = NC * NS
B = 80
ET = E // NW
NB = ET // B
RPT = NP // NS


def _spmm_body(x_hbm, rows_hbm, cols_hbm, vals_hbm, out_hbm,
               acc, cols_v, vals_v, gbuf, rbuf, sem):
    cid = lax.axis_index("c")
    sid = lax.axis_index("s")
    wid = cid * NS + sid
    ebase = wid * ET

    def zrow(j, c_):
        for c in range(D // 16):
            gbuf[j, pl.ds(c * 16, 16)] = jnp.zeros((16,), jnp.float32)
        return c_
    lax.fori_loop(0, B, zrow, 0)
    for k in range(RPT // B):
        pltpu.sync_copy(gbuf, acc.at[pl.ds(sid * RPT + k * B, B)])

    pltpu.sync_copy(cols_hbm.at[pl.ds(ebase, ET)], cols_v)
    pltpu.sync_copy(vals_hbm.at[pl.ds(ebase, ET)], vals_v.at[pl.ds(0, ET)])

    plsc.subcore_barrier()

    def batch(i, c_):
        pltpu.sync_copy(rows_hbm.at[pl.ds(ebase + i * B, B)], rbuf)
        pltpu.async_copy(x_hbm.at[cols_v.at[pl.ds(i * B, B)]], gbuf, sem).wait()

        def scale(j, cc_):
            vv = vals_v[pl.ds(i * B + j, 16)]
            v = vv[0]
            for c in range(D // 16):
                sl = pl.ds(c * 16, 16)
                gbuf[j, sl] = gbuf[j, sl] * v
            return cc_
        lax.fori_loop(0, B, scale, 0)

        pltpu.sync_copy(gbuf, acc.at[rbuf], add=True)
        return c_
    lax.fori_loop(0, NB, batch, 0)

    plsc.subcore_barrier()

    sl = pl.ds(sid * RPT, RPT)
    pltpu.sync_copy(acc.at[sl], out_hbm.at[cid, sl])


@functools.cache
def _build_spmm():
    return pl.kernel(
        _spmm_body,
        out_type=jax.ShapeDtypeStruct((NC, NP, D), jnp.float32),
        mesh=plsc.VectorSubcoreMesh(
            core_axis_name="c", subcore_axis_name="s",
            num_cores=NC, num_subcores=NS),
        scratch_types=[
            pltpu.VMEM_SHARED((NP, D), jnp.float32),
            pltpu.VMEM((ET,), jnp.int32),
            pltpu.VMEM((ET + 16,), jnp.float32),
            pltpu.VMEM((B, D), jnp.float32),
            pltpu.VMEM((B,), jnp.int32),
            pltpu.SemaphoreType.DMA,
        ],
    )


BM = 1024


def _mm_body(p_ref, w_ref, o_ref):
    x = p_ref[0] + p_ref[1]
    o_ref[...] = lax.dot_general(
        x, w_ref[...], (((1,), (1,)), ((), ())),
        preferred_element_type=jnp.float32)


_mm = pl.pallas_call(
    _mm_body,
    grid=(NP // BM,),
    in_specs=[
        pl.BlockSpec((NC, BM, D), lambda i: (0, i, 0)),
        pl.BlockSpec((D, D), lambda i: (0, 0)),
    ],
    out_specs=pl.BlockSpec((BM, D), lambda i: (i, 0)),
    out_shape=jax.ShapeDtypeStruct((N, D), jnp.float32),
)


def kernel(X, W, rows, cols, vals):
    partials = _build_spmm()(X, rows.astype(jnp.int32), cols.astype(jnp.int32),
                             vals)
    return _mm(partials, W)
